# baseline (speedup 1.0000x reference)
import dataclasses
import functools

import jax
import jax.numpy as jnp
from jax import lax
from jax.experimental import pallas as pl
from jax.experimental.pallas import tpu as pltpu
from jax.experimental.pallas import tpu_sc as plsc

BATCH = 16384
FACTORS = 32
LANES = 16
NC = 2
NS = 16
NW = NC * NS
BPW = BATCH // NW
CHUNK = 128
NCHUNK = BPW // CHUNK


def _mf_body(uf_hbm, if_hbm, ui_hbm, ii_hbm, out_hbm,
             ui_v, ii_v, u_v, v_v, o_v, sem_g):
    wid = lax.axis_index("s") * NC + lax.axis_index("c")
    base = wid * BPW

    pltpu.sync_copy(ui_hbm.at[pl.ds(wid * NCHUNK, NCHUNK)], ui_v)
    pltpu.sync_copy(ii_hbm.at[pl.ds(wid * NCHUNK, NCHUNK)], ii_v)

    copies = []
    for j in range(NCHUNK):
        copies.append(pltpu.async_copy(
            uf_hbm.at[ui_v.at[j]], u_v.at[pl.ds(j * CHUNK, CHUNK)], sem_g))
        copies.append(pltpu.async_copy(
            if_hbm.at[ii_v.at[j]], v_v.at[pl.ds(j * CHUNK, CHUNK)], sem_g))
    for c in copies:
        c.wait()

    lane = lax.broadcasted_iota(jnp.int32, (LANES,), 0)

    @pl.loop(0, BPW // LANES)
    def _(g):
        t = jnp.zeros((LANES,), jnp.float32)
        for k in range(LANES):
            r = g * LANES + k
            u0 = u_v[r, pl.ds(0, LANES)]
            u1 = u_v[r, pl.ds(LANES, LANES)]
            v0 = v_v[r, pl.ds(0, LANES)]
            v1 = v_v[r, pl.ds(LANES, LANES)]
            s = jnp.sum(u0 * v0 + u1 * v1)
            t = jnp.where(lane == k, s, t)
        o_v[pl.ds(g * LANES, LANES)] = t

    pltpu.sync_copy(o_v, out_hbm.at[pl.ds(base, BPW)])


def kernel(user_idx, item_idx, user_factors, item_factors):
    uidx = user_idx.astype(jnp.int32).reshape(NW * NCHUNK, CHUNK)
    iidx = item_idx.astype(jnp.int32).reshape(NW * NCHUNK, CHUNK)

    mesh = plsc.VectorSubcoreMesh(core_axis_name="c", subcore_axis_name="s")
    cp = pltpu.CompilerParams()
    if "needs_layout_passes" in pltpu.CompilerParams.__dataclass_fields__:
        cp = dataclasses.replace(cp, needs_layout_passes=False,
                                 use_tc_tiling_on_sc=False)
    mf = functools.partial(
        pl.kernel,
        compiler_params=cp,
        out_type=jax.ShapeDtypeStruct((BATCH,), jnp.float32),
        mesh=mesh,
        scratch_types=[
            pltpu.VMEM((NCHUNK, CHUNK), jnp.int32),
            pltpu.VMEM((NCHUNK, CHUNK), jnp.int32),
            pltpu.VMEM((BPW, FACTORS), jnp.float32),
            pltpu.VMEM((BPW, FACTORS), jnp.float32),
            pltpu.VMEM((BPW,), jnp.float32),
            pltpu.SemaphoreType.DMA,
        ],
    )(_mf_body)
    return mf(user_factors, item_factors, uidx, iidx)
